# baseline (device time: 231091 ns/iter reference)
import jax
import jax.numpy as jnp
from jax import lax
from jax.experimental import pallas as pl
from jax.experimental.pallas import tpu as pltpu

N_DEV = 8
SQ = 2048
D = 1024
HQ = 8
DH = 128
SKV_LOC = 2048
BLK = 64
CHUNK = SQ // N_DEV
QT = 512
N_QT = SQ // QT
SCALE = 0.08838834764831843
NEG = -1e9


CLS = 11 * BLK
SQP = 3 * CLS


def _attn_body(x_ref, wq_ref, k_ref, v_ref, o_ref, m_ref, l_ref, kc, vc):
    p = lax.axis_index("i")
    h = pl.program_id(0)
    is_p0 = p == 0
    for r in range(3):
        seg = r * CLS
        q = jnp.dot(x_ref[0, seg:seg + CLS, :], wq_ref[:, :],
                    preferred_element_type=jnp.float32)
        c = (3 - r) % 3
        b0 = lax.rem(jnp.int32(c) - 2 * p + 48, 3)
        for j in range(11):
            start = jnp.minimum(b0 * BLK + j * 3 * BLK, SKV_LOC - BLK)
            kc[j * BLK:(j + 1) * BLK, :] = k_ref[pl.ds(start, BLK), :]
            vc[j * BLK:(j + 1) * BLK, :] = v_ref[pl.ds(start, BLK), :]
        nv = jnp.where(b0 == 2, 10 * BLK, 11 * BLK)
        s = lax.dot_general(q, kc[:, :], (((1,), (1,)), ((), ())),
                            preferred_element_type=jnp.float32) * SCALE
        col = lax.broadcasted_iota(jnp.int32, (1, CLS), 1)
        s = jnp.where(col < nv, s, NEG)
        m = jnp.max(s, axis=1, keepdims=True)
        if r == 0:
            w = jnp.exp(s - m)
            lsum = jnp.sum(w, axis=1, keepdims=True)
            o = lax.dot_general(w, vc[:, :], (((1,), (0,)), ((), ())),
                                preferred_element_type=jnp.float32)
        else:
            s0 = lax.dot_general(q, k_ref[0:BLK, :],
                                 (((1,), (1,)), ((), ())),
                                 preferred_element_type=jnp.float32) * SCALE
            s0 = jnp.where(is_p0, s0, NEG)
            q3 = q.reshape(11, BLK, DH)
            kd = jnp.stack([k_ref[min(r + 3 * j, 31) * BLK:
                                  (min(r + 3 * j, 31) + 1) * BLK, :]
                            for j in range(11)])
            vd = jnp.stack([v_ref[min(r + 3 * j, 31) * BLK:
                                  (min(r + 3 * j, 31) + 1) * BLK, :]
                            for j in range(11)])
            sd = lax.dot_general(q3, kd, (((2,), (2,)), ((0,), (0,))),
                                 preferred_element_type=jnp.float32) * SCALE
            sd = jnp.where(is_p0, sd, NEG).reshape(CLS, BLK)
            m = jnp.maximum(
                m, jnp.maximum(jnp.max(s0, axis=1, keepdims=True),
                               jnp.max(sd, axis=1, keepdims=True)))
            w = jnp.exp(s - m)
            w0 = jnp.exp(s0 - m)
            wd = jnp.exp(sd - m)
            lsum = (jnp.sum(w, axis=1, keepdims=True)
                    + jnp.sum(w0, axis=1, keepdims=True)
                    + jnp.sum(wd, axis=1, keepdims=True))
            od = lax.dot_general(wd.reshape(11, BLK, BLK), vd,
                                 (((2,), (1,)), ((0,), (0,))),
                                 preferred_element_type=jnp.float32)
            o = (lax.dot_general(w, vc[:, :], (((1,), (0,)), ((), ())),
                                 preferred_element_type=jnp.float32)
                 + lax.dot_general(w0, v_ref[0:BLK, :],
                                   (((1,), (0,)), ((), ())),
                                   preferred_element_type=jnp.float32)
                 + od.reshape(CLS, DH))
        o_ref[seg:seg + CLS, :] = o
        lane = lax.broadcasted_iota(jnp.int32, (CLS, HQ), 1)
        m_ref[seg:seg + CLS, :] = jnp.where(
            lane == h, m, m_ref[seg:seg + CLS, :])
        l_ref[seg:seg + CLS, :] = jnp.where(
            lane == h, lsum, l_ref[seg:seg + CLS, :])


def _ring_body(o_ref, m_ref, l_ref, wo_ref, out_ref,
               o_cacc, ml_cacc, o_rx, ml_rx,
               rs_ssems, rs_rsems, ag_ssems, ag_rsems):
    p = lax.axis_index("i")

    barrier = pltpu.get_barrier_semaphore()
    for d in range(1, N_DEV):
        pl.semaphore_signal(barrier, inc=1,
                            device_id=(lax.rem(p + d, N_DEV),),
                            device_id_type=pl.DeviceIdType.MESH)
    pl.semaphore_wait(barrier, N_DEV - 1)

    rdmas = []
    for d in range(1, N_DEV):
        c = lax.rem(p - d + N_DEV, N_DEV)
        o_rdma = pltpu.make_async_remote_copy(
            src_ref=o_ref.at[pl.ds(c * CHUNK, CHUNK), :],
            dst_ref=o_rx.at[d - 1],
            send_sem=rs_ssems.at[0, d],
            recv_sem=rs_rsems.at[0, d - 1],
            device_id=(c,), device_id_type=pl.DeviceIdType.MESH)
        o_rdma.start()
        rdmas.append(o_rdma)
    for d in range(1, N_DEV):
        c = lax.rem(p - d + N_DEV, N_DEV)
        m_rdma = pltpu.make_async_remote_copy(
            src_ref=m_ref.at[pl.ds(c * CHUNK, CHUNK), :],
            dst_ref=ml_rx.at[d - 1, 0],
            send_sem=rs_ssems.at[1, d],
            recv_sem=rs_rsems.at[1, d - 1],
            device_id=(c,), device_id_type=pl.DeviceIdType.MESH)
        l_rdma = pltpu.make_async_remote_copy(
            src_ref=l_ref.at[pl.ds(c * CHUNK, CHUNK), :],
            dst_ref=ml_rx.at[d - 1, 1],
            send_sem=rs_ssems.at[2, d],
            recv_sem=rs_rsems.at[2, d - 1],
            device_id=(c,), device_id_type=pl.DeviceIdType.MESH)
        m_rdma.start()
        l_rdma.start()
        rdmas += [m_rdma, l_rdma]

    own = pl.ds(p * CHUNK, CHUNK)
    o_cacc[:, :] = o_ref[own, :]
    ml_cacc[0, :, :] = m_ref[own, :]
    ml_cacc[1, :, :] = l_ref[own, :]

    for j in range(N_DEV - 1):
        rdmas[j].wait_recv()
        rdmas[7 + 2 * j].wait_recv()
        rdmas[8 + 2 * j].wait_recv()
        m_loc = ml_cacc[0, :, :]
        l_loc = ml_cacc[1, :, :]
        m_rx_v = ml_rx[j, 0]
        l_rx_v = ml_rx[j, 1]
        m_new = jnp.maximum(m_loc, m_rx_v)
        a_loc = jnp.exp(m_loc - m_new)
        a_rx = jnp.exp(m_rx_v - m_new)
        for h in range(HQ):
            cols = slice(h * DH, (h + 1) * DH)
            o_cacc[:, cols] = (
                o_cacc[:, cols] * a_loc[:, h:h + 1]
                + o_rx[j][:, cols] * a_rx[:, h:h + 1])
        ml_cacc[0, :, :] = m_new
        ml_cacc[1, :, :] = l_loc * a_loc + l_rx_v * a_rx

    l_own = ml_cacc[1, :, :]
    ctx_cols = []
    for h in range(HQ):
        cols = slice(h * DH, (h + 1) * DH)
        ctx_cols.append(o_cacc[:, cols] / l_own[:, h:h + 1])
    ctx = jnp.concatenate(ctx_cols, axis=1)
    out_ref[0, own, :] = jnp.dot(ctx, wo_ref[:, :],
                                 preferred_element_type=jnp.float32)

    for d in range(1, N_DEV):
        tgt = lax.rem(p + d, N_DEV)
        rdma = pltpu.make_async_remote_copy(
            src_ref=out_ref.at[0, own, :],
            dst_ref=out_ref.at[0, own, :],
            send_sem=ag_ssems.at[d],
            recv_sem=ag_rsems.at[p],
            device_id=(tgt,), device_id_type=pl.DeviceIdType.MESH)
        rdma.start()
        rdmas.append(rdma)
    for d in range(1, N_DEV):
        c = lax.rem(p + d, N_DEV)
        rx = pltpu.make_async_remote_copy(
            src_ref=out_ref.at[0, pl.ds(c * CHUNK, CHUNK), :],
            dst_ref=out_ref.at[0, pl.ds(c * CHUNK, CHUNK), :],
            send_sem=ag_ssems.at[0],
            recv_sem=ag_rsems.at[c],
            device_id=(c,), device_id_type=pl.DeviceIdType.MESH)
        rx.wait_recv()

    for r in rdmas:
        r.wait_send()


_SRC = [(0, 11, 22)[b % 3] + b // 3 for b in range(SQ // BLK)]


def kernel(x, Wq, K_ext, V_ext, Wo):
    k2 = K_ext.reshape(SKV_LOC, HQ * DH)
    v2 = V_ext.reshape(SKV_LOC, HQ * DH)
    x4 = x.reshape(1, SQ // BLK, BLK, D)
    xp = jnp.concatenate(
        [x4[:, 0::3], x4[:, 1::3], x4[:, 2::3], x4[:, -1:]],
        axis=1).reshape(1, SQP, D)
    o, m, l = pl.pallas_call(
        _attn_body,
        grid=(HQ,),
        in_specs=[
            pl.BlockSpec((1, SQP, D), lambda h: (0, 0, 0)),
            pl.BlockSpec((D, DH), lambda h: (0, h)),
            pl.BlockSpec((SKV_LOC, DH), lambda h: (0, h)),
            pl.BlockSpec((SKV_LOC, DH), lambda h: (0, h)),
        ],
        out_specs=[
            pl.BlockSpec((SQP, DH), lambda h: (0, h)),
            pl.BlockSpec((SQP, HQ), lambda h: (0, 0)),
            pl.BlockSpec((SQP, HQ), lambda h: (0, 0)),
        ],
        out_shape=[
            jax.ShapeDtypeStruct((SQP, D), jnp.float32),
            jax.ShapeDtypeStruct((SQP, HQ), jnp.float32),
            jax.ShapeDtypeStruct((SQP, HQ), jnp.float32),
        ],
        scratch_shapes=[
            pltpu.VMEM((CLS, DH), jnp.float32),
            pltpu.VMEM((CLS, DH), jnp.float32),
        ],
    )(xp, Wq, k2, v2)

    out_perm = pl.pallas_call(
        _ring_body,
        out_shape=jax.ShapeDtypeStruct((1, SQ, D), jnp.float32),
        in_specs=[pl.BlockSpec(memory_space=pltpu.VMEM)] * 4,
        out_specs=pl.BlockSpec(memory_space=pltpu.VMEM),
        scratch_shapes=[
            pltpu.VMEM((CHUNK, D), jnp.float32),
            pltpu.VMEM((2, CHUNK, HQ), jnp.float32),
            pltpu.VMEM((N_DEV - 1, CHUNK, D), jnp.float32),
            pltpu.VMEM((N_DEV - 1, 2, CHUNK, HQ), jnp.float32),
            pltpu.SemaphoreType.DMA((3, N_DEV)),
            pltpu.SemaphoreType.DMA((3, N_DEV)),
            pltpu.SemaphoreType.DMA((N_DEV,)),
            pltpu.SemaphoreType.DMA((N_DEV,)),
        ],
        compiler_params=pltpu.CompilerParams(collective_id=0),
    )(o, m, l, Wo)
    o4 = out_perm.reshape(1, SQ // BLK, BLK, D)
    return jnp.concatenate([o4[:, s:s + 1] for s in _SRC],
                           axis=1).reshape(1, SQ, D)


# device time: 171132 ns/iter; 1.3504x vs baseline; 1.3504x over previous
import jax
import jax.numpy as jnp
from jax import lax
from jax.experimental import pallas as pl
from jax.experimental.pallas import tpu as pltpu

N_DEV = 8
SQ = 2048
D = 1024
HQ = 8
DH = 128
SKV_LOC = 2048
BLK = 64
CHUNK = SQ // N_DEV
QT = 512
N_QT = SQ // QT
SCALE = 0.08838834764831843
NEG = -1e9


CLS = 11 * BLK
SQP = 3 * CLS


def _attn_body(x_ref, wq_ref, k_ref, v_ref, o_ref, m_ref, l_ref, kc, vc):
    p = lax.axis_index("i")
    h = pl.program_id(0)
    is_p0 = p == 0
    for r in range(3):
        seg = r * CLS
        q = jnp.dot(x_ref[0, seg:seg + CLS, :], wq_ref[:, :],
                    preferred_element_type=jnp.float32)
        c = (3 - r) % 3
        b0 = lax.rem(jnp.int32(c) - 2 * p + 48, 3)
        for j in range(11):
            start = jnp.minimum(b0 * BLK + j * 3 * BLK, SKV_LOC - BLK)
            kc[j * BLK:(j + 1) * BLK, :] = k_ref[pl.ds(start, BLK), :]
            vc[j * BLK:(j + 1) * BLK, :] = v_ref[pl.ds(start, BLK), :]
        nv = jnp.where(b0 == 2, 10 * BLK, 11 * BLK)
        s = lax.dot_general(q, kc[:, :], (((1,), (1,)), ((), ())),
                            preferred_element_type=jnp.float32) * SCALE
        col = lax.broadcasted_iota(jnp.int32, (1, CLS), 1)
        s = jnp.where(col < nv, s, NEG)
        m = jnp.max(s, axis=1, keepdims=True)
        if r == 0:
            w = jnp.exp(s - m)
            lsum = jnp.sum(w, axis=1, keepdims=True)
            o = lax.dot_general(w, vc[:, :], (((1,), (0,)), ((), ())),
                                preferred_element_type=jnp.float32)
        else:
            s0 = lax.dot_general(q, k_ref[0:BLK, :],
                                 (((1,), (1,)), ((), ())),
                                 preferred_element_type=jnp.float32) * SCALE
            s0 = jnp.where(is_p0, s0, NEG)
            q3 = q.reshape(11, BLK, DH)
            kd = jnp.stack([k_ref[min(r + 3 * j, 31) * BLK:
                                  (min(r + 3 * j, 31) + 1) * BLK, :]
                            for j in range(11)])
            vd = jnp.stack([v_ref[min(r + 3 * j, 31) * BLK:
                                  (min(r + 3 * j, 31) + 1) * BLK, :]
                            for j in range(11)])
            sd = lax.dot_general(q3, kd, (((2,), (2,)), ((0,), (0,))),
                                 preferred_element_type=jnp.float32) * SCALE
            sd = jnp.where(is_p0, sd, NEG).reshape(CLS, BLK)
            m = jnp.maximum(
                m, jnp.maximum(jnp.max(s0, axis=1, keepdims=True),
                               jnp.max(sd, axis=1, keepdims=True)))
            w = jnp.exp(s - m)
            w0 = jnp.exp(s0 - m)
            wd = jnp.exp(sd - m)
            lsum = (jnp.sum(w, axis=1, keepdims=True)
                    + jnp.sum(w0, axis=1, keepdims=True)
                    + jnp.sum(wd, axis=1, keepdims=True))
            od = lax.dot_general(wd.reshape(11, BLK, BLK), vd,
                                 (((2,), (1,)), ((0,), (0,))),
                                 preferred_element_type=jnp.float32)
            o = (lax.dot_general(w, vc[:, :], (((1,), (0,)), ((), ())),
                                 preferred_element_type=jnp.float32)
                 + lax.dot_general(w0, v_ref[0:BLK, :],
                                   (((1,), (0,)), ((), ())),
                                   preferred_element_type=jnp.float32)
                 + od.reshape(CLS, DH))
        o_ref[seg:seg + CLS, :] = o
        lane = lax.broadcasted_iota(jnp.int32, (CLS, HQ), 1)
        m_ref[seg:seg + CLS, :] = jnp.where(
            lane == h, m, m_ref[seg:seg + CLS, :])
        l_ref[seg:seg + CLS, :] = jnp.where(
            lane == h, lsum, l_ref[seg:seg + CLS, :])


HALF = CHUNK // 2


def _merge(o_acc, ml_acc, o_rx_ref, ml_rx_ref, rows):
    m_loc = ml_acc[0, rows, :]
    l_loc = ml_acc[1, rows, :]
    m_rx = ml_rx_ref[0]
    l_rx = ml_rx_ref[1]
    m_new = jnp.maximum(m_loc, m_rx)
    a_loc = jnp.exp(m_loc - m_new)
    a_rx = jnp.exp(m_rx - m_new)
    for h in range(HQ):
        cols = slice(h * DH, (h + 1) * DH)
        o_acc[rows, cols] = (
            o_acc[rows, cols] * a_loc[:, h:h + 1]
            + o_rx_ref[:, cols].astype(jnp.float32) * a_rx[:, h:h + 1])
    ml_acc[0, rows, :] = m_new
    ml_acc[1, rows, :] = l_loc * a_loc + l_rx * a_rx


def _ring_body(o_ref, m_ref, l_ref, wo_ref, out_ref,
               o_acc, ml_acc, o_cw, ml_cw, o_ccw, ml_ccw,
               o_tx_cw, o_tx_ccw, ag_buf,
               cw_send_sems, cw_recv_sems, ccw_send_sems, ccw_recv_sems,
               ag_send_sems, ag_recv_sems):
    p = lax.axis_index("i")
    left = lax.rem(p - 1 + N_DEV, N_DEV)
    right = lax.rem(p + 1, N_DEV)

    barrier = pltpu.get_barrier_semaphore()
    for nbr in (left, right):
        pl.semaphore_signal(barrier, inc=1, device_id=(nbr,),
                            device_id_type=pl.DeviceIdType.MESH)
    pl.semaphore_wait(barrier, 2)

    o_acc[:, :] = o_ref[0:SQ, :]
    ml_acc[0, :, :] = m_ref[0:SQ, :]
    ml_acc[1, :, :] = l_ref[0:SQ, :]

    rdmas = []
    for s_hop in range(N_DEV - 1):
        cw_send = lax.rem(p - s_hop + N_DEV, N_DEV)
        cw_recv = lax.rem(p - s_hop - 1 + N_DEV, N_DEV)
        ccw_send = lax.rem(p + s_hop, N_DEV)
        ccw_recv = lax.rem(p + s_hop + 1, N_DEV)
        hop = []
        for (c_send, dev, o_rs, o_tx, ml_rs, ssems, rsems, off) in (
                (cw_send, right, o_cw, o_tx_cw, ml_cw,
                 cw_send_sems, cw_recv_sems, 0),
                (ccw_send, left, o_ccw, o_tx_ccw, ml_ccw,
                 ccw_send_sems, ccw_recv_sems, HALF)):
            o_tx[s_hop, :, :] = o_acc[
                pl.ds(c_send * CHUNK + off, HALF), :].astype(jnp.bfloat16)
            o_rdma = pltpu.make_async_remote_copy(
                src_ref=o_tx.at[s_hop],
                dst_ref=o_rs.at[s_hop],
                send_sem=ssems.at[0, s_hop],
                recv_sem=rsems.at[0, s_hop],
                device_id=(dev,), device_id_type=pl.DeviceIdType.MESH)
            ml_rdma = pltpu.make_async_remote_copy(
                src_ref=ml_acc.at[:, pl.ds(c_send * CHUNK + off, HALF), :],
                dst_ref=ml_rs.at[s_hop],
                send_sem=ssems.at[1, s_hop],
                recv_sem=rsems.at[1, s_hop],
                device_id=(dev,), device_id_type=pl.DeviceIdType.MESH)
            o_rdma.start()
            ml_rdma.start()
            hop += [o_rdma, ml_rdma]
        rdmas += hop
        hop[0].wait_recv()
        hop[1].wait_recv()
        _merge(o_acc, ml_acc, o_cw.at[s_hop], ml_cw.at[s_hop],
               pl.ds(cw_recv * CHUNK, HALF))
        hop[2].wait_recv()
        hop[3].wait_recv()
        _merge(o_acc, ml_acc, o_ccw.at[s_hop], ml_ccw.at[s_hop],
               pl.ds(ccw_recv * CHUNK + HALF, HALF))

    for (c_own, off) in ((lax.rem(p + 1, N_DEV), 0),
                         (lax.rem(p - 1 + N_DEV, N_DEV), HALF)):
        rows = pl.ds(c_own * CHUNK + off, HALF)
        l_own = ml_acc[1, rows, :]
        ctx_cols = []
        for h in range(HQ):
            cols = slice(h * DH, (h + 1) * DH)
            ctx_cols.append(o_acc[rows, cols] / l_own[:, h:h + 1])
        ctx = jnp.concatenate(ctx_cols, axis=1)
        ag_buf[rows, :] = jnp.dot(
            ctx, wo_ref[:, :],
            preferred_element_type=jnp.float32).astype(jnp.bfloat16)

    for h_hop in range(N_DEV - 1):
        cw_c = lax.rem(p + 1 - h_hop + N_DEV, N_DEV)
        ccw_c = lax.rem(p - 1 + h_hop + N_DEV, N_DEV)
        hop = []
        for (c, dev, ssems, rsems, off) in (
                (cw_c, right, ag_send_sems, ag_recv_sems, 0),
                (ccw_c, left, ag_send_sems, ag_recv_sems, HALF)):
            src = ag_buf.at[pl.ds(c * CHUNK + off, HALF), :]
            rdma = pltpu.make_async_remote_copy(
                src_ref=src, dst_ref=src,
                send_sem=ssems.at[0 if off == 0 else 1, h_hop],
                recv_sem=rsems.at[0 if off == 0 else 1, h_hop],
                device_id=(dev,), device_id_type=pl.DeviceIdType.MESH)
            rdma.start()
            hop.append(rdma)
        hop[0].wait_recv()
        hop[1].wait_recv()
        rdmas += hop

    out_ref[0, :, :] = ag_buf[:, :].astype(jnp.float32)

    for r in rdmas:
        r.wait_send()


_SRC = [(0, 11, 22)[b % 3] + b // 3 for b in range(SQ // BLK)]


def kernel(x, Wq, K_ext, V_ext, Wo):
    k2 = K_ext.reshape(SKV_LOC, HQ * DH)
    v2 = V_ext.reshape(SKV_LOC, HQ * DH)
    x4 = x.reshape(1, SQ // BLK, BLK, D)
    xp = jnp.concatenate(
        [x4[:, 0::3], x4[:, 1::3], x4[:, 2::3], x4[:, -1:]],
        axis=1).reshape(1, SQP, D)
    o, m, l = pl.pallas_call(
        _attn_body,
        grid=(HQ,),
        in_specs=[
            pl.BlockSpec((1, SQP, D), lambda h: (0, 0, 0)),
            pl.BlockSpec((D, DH), lambda h: (0, h)),
            pl.BlockSpec((SKV_LOC, DH), lambda h: (0, h)),
            pl.BlockSpec((SKV_LOC, DH), lambda h: (0, h)),
        ],
        out_specs=[
            pl.BlockSpec((SQP, DH), lambda h: (0, h)),
            pl.BlockSpec((SQP, HQ), lambda h: (0, 0)),
            pl.BlockSpec((SQP, HQ), lambda h: (0, 0)),
        ],
        out_shape=[
            jax.ShapeDtypeStruct((SQP, D), jnp.float32),
            jax.ShapeDtypeStruct((SQP, HQ), jnp.float32),
            jax.ShapeDtypeStruct((SQP, HQ), jnp.float32),
        ],
        scratch_shapes=[
            pltpu.VMEM((CLS, DH), jnp.float32),
            pltpu.VMEM((CLS, DH), jnp.float32),
        ],
    )(xp, Wq, k2, v2)

    out_perm = pl.pallas_call(
        _ring_body,
        out_shape=jax.ShapeDtypeStruct((1, SQ, D), jnp.float32),
        in_specs=[pl.BlockSpec(memory_space=pltpu.VMEM)] * 4,
        out_specs=pl.BlockSpec(memory_space=pltpu.VMEM),
        scratch_shapes=[
            pltpu.VMEM((SQ, D), jnp.float32),
            pltpu.VMEM((2, SQ, HQ), jnp.float32),
            pltpu.VMEM((N_DEV - 1, HALF, D), jnp.bfloat16),
            pltpu.VMEM((N_DEV - 1, 2, HALF, HQ), jnp.float32),
            pltpu.VMEM((N_DEV - 1, HALF, D), jnp.bfloat16),
            pltpu.VMEM((N_DEV - 1, 2, HALF, HQ), jnp.float32),
            pltpu.VMEM((N_DEV - 1, HALF, D), jnp.bfloat16),
            pltpu.VMEM((N_DEV - 1, HALF, D), jnp.bfloat16),
            pltpu.VMEM((SQ, D), jnp.bfloat16),
            pltpu.SemaphoreType.DMA((2, N_DEV - 1)),
            pltpu.SemaphoreType.DMA((2, N_DEV - 1)),
            pltpu.SemaphoreType.DMA((2, N_DEV - 1)),
            pltpu.SemaphoreType.DMA((2, N_DEV - 1)),
            pltpu.SemaphoreType.DMA((2, N_DEV - 1)),
            pltpu.SemaphoreType.DMA((2, N_DEV - 1)),
        ],
        compiler_params=pltpu.CompilerParams(collective_id=0),
    )(o, m, l, Wo)
    o4 = out_perm.reshape(1, SQ // BLK, BLK, D)
    return jnp.concatenate([o4[:, s:s + 1] for s in _SRC],
                           axis=1).reshape(1, SQ, D)


# device time: 166710 ns/iter; 1.3862x vs baseline; 1.0265x over previous
import jax
import jax.numpy as jnp
from jax import lax
from jax.experimental import pallas as pl
from jax.experimental.pallas import tpu as pltpu

N_DEV = 8
SQ = 2048
D = 1024
HQ = 8
DH = 128
SKV_LOC = 2048
BLK = 64
CHUNK = SQ // N_DEV
QT = 512
N_QT = SQ // QT
SCALE = 0.08838834764831843
NEG = -1e9


CLS = 11 * BLK
SQP = 3 * CLS


def _attn_body(x_ref, wq_ref, k_ref, v_ref, o_ref, m_ref, l_ref, kc, vc):
    p = lax.axis_index("i")
    h = pl.program_id(0)
    is_p0 = p == 0
    for r in range(3):
        seg = r * CLS
        q = jnp.dot(x_ref[0, seg:seg + CLS, :], wq_ref[:, :],
                    preferred_element_type=jnp.float32)
        c = (3 - r) % 3
        b0 = lax.rem(jnp.int32(c) - 2 * p + 48, 3)
        for j in range(11):
            start = jnp.minimum(b0 * BLK + j * 3 * BLK, SKV_LOC - BLK)
            kc[j * BLK:(j + 1) * BLK, :] = k_ref[pl.ds(start, BLK), :]
            vc[j * BLK:(j + 1) * BLK, :] = v_ref[pl.ds(start, BLK), :]
        nv = jnp.where(b0 == 2, 10 * BLK, 11 * BLK)
        s = lax.dot_general(q, kc[:, :], (((1,), (1,)), ((), ())),
                            preferred_element_type=jnp.float32) * SCALE
        col = lax.broadcasted_iota(jnp.int32, (1, CLS), 1)
        s = jnp.where(col < nv, s, NEG)
        m = jnp.max(s, axis=1, keepdims=True)
        if r == 0:
            w = jnp.exp(s - m)
            lsum = jnp.sum(w, axis=1, keepdims=True)
            o = lax.dot_general(w, vc[:, :], (((1,), (0,)), ((), ())),
                                preferred_element_type=jnp.float32)
        else:
            s0 = lax.dot_general(q, k_ref[0:BLK, :],
                                 (((1,), (1,)), ((), ())),
                                 preferred_element_type=jnp.float32) * SCALE
            s0 = jnp.where(is_p0, s0, NEG)
            q3 = q.reshape(11, BLK, DH)
            kd = jnp.stack([k_ref[min(r + 3 * j, 31) * BLK:
                                  (min(r + 3 * j, 31) + 1) * BLK, :]
                            for j in range(11)])
            vd = jnp.stack([v_ref[min(r + 3 * j, 31) * BLK:
                                  (min(r + 3 * j, 31) + 1) * BLK, :]
                            for j in range(11)])
            sd = lax.dot_general(q3, kd, (((2,), (2,)), ((0,), (0,))),
                                 preferred_element_type=jnp.float32) * SCALE
            sd = jnp.where(is_p0, sd, NEG).reshape(CLS, BLK)
            m = jnp.maximum(
                m, jnp.maximum(jnp.max(s0, axis=1, keepdims=True),
                               jnp.max(sd, axis=1, keepdims=True)))
            w = jnp.exp(s - m)
            w0 = jnp.exp(s0 - m)
            wd = jnp.exp(sd - m)
            lsum = (jnp.sum(w, axis=1, keepdims=True)
                    + jnp.sum(w0, axis=1, keepdims=True)
                    + jnp.sum(wd, axis=1, keepdims=True))
            od = lax.dot_general(wd.reshape(11, BLK, BLK), vd,
                                 (((2,), (1,)), ((0,), (0,))),
                                 preferred_element_type=jnp.float32)
            o = (lax.dot_general(w, vc[:, :], (((1,), (0,)), ((), ())),
                                 preferred_element_type=jnp.float32)
                 + lax.dot_general(w0, v_ref[0:BLK, :],
                                   (((1,), (0,)), ((), ())),
                                   preferred_element_type=jnp.float32)
                 + od.reshape(CLS, DH))
        o_ref[seg:seg + CLS, :] = o.astype(jnp.bfloat16)
        lane = lax.broadcasted_iota(jnp.int32, (CLS, HQ), 1)
        m_ref[seg:seg + CLS, :] = jnp.where(
            lane == h, m, m_ref[seg:seg + CLS, :])
        l_ref[seg:seg + CLS, :] = jnp.where(
            lane == h, lsum, l_ref[seg:seg + CLS, :])


HALF = CHUNK // 2


def _merge(o_acc, ml_acc, o_rx_ref, ml_rx_ref, rows):
    m_loc = ml_acc[0, rows, :]
    l_loc = ml_acc[1, rows, :]
    m_rx = ml_rx_ref[0]
    l_rx = ml_rx_ref[1]
    m_new = jnp.maximum(m_loc, m_rx)
    a_loc = jnp.exp(m_loc - m_new)
    a_rx = jnp.exp(m_rx - m_new)
    for h in range(HQ):
        cols = slice(h * DH, (h + 1) * DH)
        o_acc[rows, cols] = (
            o_acc[rows, cols] * a_loc[:, h:h + 1]
            + o_rx_ref[:, cols].astype(jnp.float32) * a_rx[:, h:h + 1])
    ml_acc[0, rows, :] = m_new
    ml_acc[1, rows, :] = l_loc * a_loc + l_rx * a_rx


def _ring_body(o_ref, m_ref, l_ref, wo_ref, out_ref,
               o_acc, ml_acc, o_cw, ml_cw, o_ccw, ml_ccw,
               o_tx_cw, o_tx_ccw, ag_buf,
               cw_send_sems, cw_recv_sems, ccw_send_sems, ccw_recv_sems,
               ag_send_sems, ag_recv_sems):
    p = lax.axis_index("i")
    left = lax.rem(p - 1 + N_DEV, N_DEV)
    right = lax.rem(p + 1, N_DEV)

    barrier = pltpu.get_barrier_semaphore()
    for d in range(1, N_DEV):
        pl.semaphore_signal(barrier, inc=1,
                            device_id=(lax.rem(p + d, N_DEV),),
                            device_id_type=pl.DeviceIdType.MESH)
    pl.semaphore_wait(barrier, N_DEV - 1)

    o_acc[:, :] = o_ref[0:SQ, :].astype(jnp.float32)
    ml_acc[0, :, :] = m_ref[0:SQ, :]
    ml_acc[1, :, :] = l_ref[0:SQ, :]

    rdmas = []
    for s_hop in range(N_DEV - 1):
        cw_send = lax.rem(p - s_hop + N_DEV, N_DEV)
        cw_recv = lax.rem(p - s_hop - 1 + N_DEV, N_DEV)
        ccw_send = lax.rem(p + s_hop, N_DEV)
        ccw_recv = lax.rem(p + s_hop + 1, N_DEV)
        hop = []
        for (c_send, dev, o_rs, o_tx, ml_rs, ssems, rsems, off) in (
                (cw_send, right, o_cw, o_tx_cw, ml_cw,
                 cw_send_sems, cw_recv_sems, 0),
                (ccw_send, left, o_ccw, o_tx_ccw, ml_ccw,
                 ccw_send_sems, ccw_recv_sems, HALF)):
            o_tx[s_hop, :, :] = o_acc[
                pl.ds(c_send * CHUNK + off, HALF), :].astype(jnp.bfloat16)
            o_rdma = pltpu.make_async_remote_copy(
                src_ref=o_tx.at[s_hop],
                dst_ref=o_rs.at[s_hop],
                send_sem=ssems.at[0, s_hop],
                recv_sem=rsems.at[0, s_hop],
                device_id=(dev,), device_id_type=pl.DeviceIdType.MESH)
            ml_rdma = pltpu.make_async_remote_copy(
                src_ref=ml_acc.at[:, pl.ds(c_send * CHUNK + off, HALF), :],
                dst_ref=ml_rs.at[s_hop],
                send_sem=ssems.at[1, s_hop],
                recv_sem=rsems.at[1, s_hop],
                device_id=(dev,), device_id_type=pl.DeviceIdType.MESH)
            o_rdma.start()
            ml_rdma.start()
            hop += [o_rdma, ml_rdma]
        rdmas += hop
        hop[0].wait_recv()
        hop[1].wait_recv()
        _merge(o_acc, ml_acc, o_cw.at[s_hop], ml_cw.at[s_hop],
               pl.ds(cw_recv * CHUNK, HALF))
        hop[2].wait_recv()
        hop[3].wait_recv()
        _merge(o_acc, ml_acc, o_ccw.at[s_hop], ml_ccw.at[s_hop],
               pl.ds(ccw_recv * CHUNK + HALF, HALF))

    for (c_own, off) in ((lax.rem(p + 1, N_DEV), 0),
                         (lax.rem(p - 1 + N_DEV, N_DEV), HALF)):
        rows = pl.ds(c_own * CHUNK + off, HALF)
        l_own = ml_acc[1, rows, :]
        ctx_cols = []
        for h in range(HQ):
            cols = slice(h * DH, (h + 1) * DH)
            ctx_cols.append(o_acc[rows, cols] / l_own[:, h:h + 1])
        ctx = jnp.concatenate(ctx_cols, axis=1)
        ag_buf[rows, :] = jnp.dot(
            ctx, wo_ref[:, :],
            preferred_element_type=jnp.float32).astype(jnp.bfloat16)

    c_top = lax.rem(p + 1, N_DEV)
    c_bot = lax.rem(p - 1 + N_DEV, N_DEV)
    for d in range(1, N_DEV):
        tgt = lax.rem(p + d, N_DEV)
        for (kind, c, off) in ((0, c_top, 0), (1, c_bot, HALF)):
            src = ag_buf.at[pl.ds(c * CHUNK + off, HALF), :]
            rdma = pltpu.make_async_remote_copy(
                src_ref=src, dst_ref=src,
                send_sem=ag_send_sems.at[kind, d],
                recv_sem=ag_recv_sems.at[kind, c],
                device_id=(tgt,), device_id_type=pl.DeviceIdType.MESH)
            rdma.start()
            rdmas.append(rdma)
    for d in range(1, N_DEV):
        for (kind, c, off) in (
                (0, lax.rem(p + 1 + d, N_DEV), 0),
                (1, lax.rem(p - 1 + d, N_DEV), HALF)):
            dst = ag_buf.at[pl.ds(c * CHUNK + off, HALF), :]
            rx = pltpu.make_async_remote_copy(
                src_ref=dst, dst_ref=dst,
                send_sem=ag_send_sems.at[kind, 0],
                recv_sem=ag_recv_sems.at[kind, c],
                device_id=(p,), device_id_type=pl.DeviceIdType.MESH)
            rx.wait_recv()

    out_ref[0, :, :] = ag_buf[:, :].astype(jnp.float32)

    for r in rdmas:
        r.wait_send()


_SRC = [(0, 11, 22)[b % 3] + b // 3 for b in range(SQ // BLK)]


def kernel(x, Wq, K_ext, V_ext, Wo):
    k2 = K_ext.reshape(SKV_LOC, HQ * DH)
    v2 = V_ext.reshape(SKV_LOC, HQ * DH)
    x4 = x.reshape(1, SQ // BLK, BLK, D)
    xp = jnp.concatenate(
        [x4[:, 0::3], x4[:, 1::3], x4[:, 2::3], x4[:, -1:]],
        axis=1).reshape(1, SQP, D)
    o, m, l = pl.pallas_call(
        _attn_body,
        grid=(HQ,),
        in_specs=[
            pl.BlockSpec((1, SQP, D), lambda h: (0, 0, 0)),
            pl.BlockSpec((D, DH), lambda h: (0, h)),
            pl.BlockSpec((SKV_LOC, DH), lambda h: (0, h)),
            pl.BlockSpec((SKV_LOC, DH), lambda h: (0, h)),
        ],
        out_specs=[
            pl.BlockSpec((SQP, DH), lambda h: (0, h)),
            pl.BlockSpec((SQP, HQ), lambda h: (0, 0)),
            pl.BlockSpec((SQP, HQ), lambda h: (0, 0)),
        ],
        out_shape=[
            jax.ShapeDtypeStruct((SQP, D), jnp.bfloat16),
            jax.ShapeDtypeStruct((SQP, HQ), jnp.float32),
            jax.ShapeDtypeStruct((SQP, HQ), jnp.float32),
        ],
        scratch_shapes=[
            pltpu.VMEM((CLS, DH), jnp.float32),
            pltpu.VMEM((CLS, DH), jnp.float32),
        ],
    )(xp, Wq, k2, v2)

    out_perm = pl.pallas_call(
        _ring_body,
        out_shape=jax.ShapeDtypeStruct((1, SQ, D), jnp.float32),
        in_specs=[pl.BlockSpec(memory_space=pltpu.VMEM)] * 4,
        out_specs=pl.BlockSpec(memory_space=pltpu.VMEM),
        scratch_shapes=[
            pltpu.VMEM((SQ, D), jnp.float32),
            pltpu.VMEM((2, SQ, HQ), jnp.float32),
            pltpu.VMEM((N_DEV - 1, HALF, D), jnp.bfloat16),
            pltpu.VMEM((N_DEV - 1, 2, HALF, HQ), jnp.float32),
            pltpu.VMEM((N_DEV - 1, HALF, D), jnp.bfloat16),
            pltpu.VMEM((N_DEV - 1, 2, HALF, HQ), jnp.float32),
            pltpu.VMEM((N_DEV - 1, HALF, D), jnp.bfloat16),
            pltpu.VMEM((N_DEV - 1, HALF, D), jnp.bfloat16),
            pltpu.VMEM((SQ, D), jnp.bfloat16),
            pltpu.SemaphoreType.DMA((2, N_DEV - 1)),
            pltpu.SemaphoreType.DMA((2, N_DEV - 1)),
            pltpu.SemaphoreType.DMA((2, N_DEV - 1)),
            pltpu.SemaphoreType.DMA((2, N_DEV - 1)),
            pltpu.SemaphoreType.DMA((2, N_DEV)),
            pltpu.SemaphoreType.DMA((2, N_DEV)),
        ],
        compiler_params=pltpu.CompilerParams(collective_id=0),
    )(o, m, l, Wo)
    o4 = out_perm.reshape(1, SQ // BLK, BLK, D)
    return jnp.concatenate([o4[:, s:s + 1] for s in _SRC],
                           axis=1).reshape(1, SQ, D)


# device time: 165537 ns/iter; 1.3960x vs baseline; 1.0071x over previous
import jax
import jax.numpy as jnp
from jax import lax
from jax.experimental import pallas as pl
from jax.experimental.pallas import tpu as pltpu

N_DEV = 8
SQ = 2048
D = 1024
HQ = 8
DH = 128
SKV_LOC = 2048
BLK = 64
CHUNK = SQ // N_DEV
QT = 512
N_QT = SQ // QT
SCALE = 0.08838834764831843
NEG = -1e9


CLS = 11 * BLK
SQP = 3 * CLS


def _attn_body(x_ref, wq_ref, k_ref, v_ref, o_ref, m_ref, l_ref, kc, vc):
    p = lax.axis_index("i")
    h = pl.program_id(0)
    is_p0 = p == 0
    for r in range(3):
        seg = r * CLS
        q = jnp.dot(x_ref[0, seg:seg + CLS, :], wq_ref[:, :],
                    preferred_element_type=jnp.float32)
        c = (3 - r) % 3
        b0 = lax.rem(jnp.int32(c) - 2 * p + 48, 3)
        for j in range(11):
            start = jnp.minimum(b0 * BLK + j * 3 * BLK, SKV_LOC - BLK)
            kc[j * BLK:(j + 1) * BLK, :] = k_ref[pl.ds(start, BLK), :]
            vc[j * BLK:(j + 1) * BLK, :] = v_ref[pl.ds(start, BLK), :]
        nv = jnp.where(b0 == 2, 10 * BLK, 11 * BLK)
        s = lax.dot_general(q, kc[:, :], (((1,), (1,)), ((), ())),
                            preferred_element_type=jnp.float32) * SCALE
        col = lax.broadcasted_iota(jnp.int32, (1, CLS), 1)
        s = jnp.where(col < nv, s, NEG)
        m = jnp.max(s, axis=1, keepdims=True)
        if r == 0:
            w = jnp.exp(s - m)
            lsum = jnp.sum(w, axis=1, keepdims=True)
            o = lax.dot_general(w, vc[:, :], (((1,), (0,)), ((), ())),
                                preferred_element_type=jnp.float32)
        else:
            s0 = lax.dot_general(q, k_ref[0:BLK, :],
                                 (((1,), (1,)), ((), ())),
                                 preferred_element_type=jnp.float32) * SCALE
            s0 = jnp.where(is_p0, s0, NEG)
            q3 = q.reshape(11, BLK, DH)
            kd = jnp.stack([k_ref[min(r + 3 * j, 31) * BLK:
                                  (min(r + 3 * j, 31) + 1) * BLK, :]
                            for j in range(11)])
            vd = jnp.stack([v_ref[min(r + 3 * j, 31) * BLK:
                                  (min(r + 3 * j, 31) + 1) * BLK, :]
                            for j in range(11)])
            sd = lax.dot_general(q3, kd, (((2,), (2,)), ((0,), (0,))),
                                 preferred_element_type=jnp.float32) * SCALE
            sd = jnp.where(is_p0, sd, NEG).reshape(CLS, BLK)
            m = jnp.maximum(
                m, jnp.maximum(jnp.max(s0, axis=1, keepdims=True),
                               jnp.max(sd, axis=1, keepdims=True)))
            w = jnp.exp(s - m)
            w0 = jnp.exp(s0 - m)
            wd = jnp.exp(sd - m)
            lsum = (jnp.sum(w, axis=1, keepdims=True)
                    + jnp.sum(w0, axis=1, keepdims=True)
                    + jnp.sum(wd, axis=1, keepdims=True))
            od = lax.dot_general(wd.reshape(11, BLK, BLK), vd,
                                 (((2,), (1,)), ((0,), (0,))),
                                 preferred_element_type=jnp.float32)
            o = (lax.dot_general(w, vc[:, :], (((1,), (0,)), ((), ())),
                                 preferred_element_type=jnp.float32)
                 + lax.dot_general(w0, v_ref[0:BLK, :],
                                   (((1,), (0,)), ((), ())),
                                   preferred_element_type=jnp.float32)
                 + od.reshape(CLS, DH))
        o_ref[seg:seg + CLS, :] = o.astype(jnp.bfloat16)
        lane = lax.broadcasted_iota(jnp.int32, (CLS, HQ), 1)
        m_ref[seg:seg + CLS, :] = jnp.where(
            lane == h, m, m_ref[seg:seg + CLS, :])
        l_ref[seg:seg + CLS, :] = jnp.where(
            lane == h, lsum, l_ref[seg:seg + CLS, :])


HALF = CHUNK // 2


def _merge(o_acc, ml_acc, o_rx_ref, ml_rx_ref, rows):
    m_loc = ml_acc[0, rows, :]
    l_loc = ml_acc[1, rows, :]
    m_rx = ml_rx_ref[0]
    l_rx = ml_rx_ref[1]
    m_new = jnp.maximum(m_loc, m_rx)
    a_loc = jnp.exp(m_loc - m_new)
    a_rx = jnp.exp(m_rx - m_new)
    for h in range(HQ):
        cols = slice(h * DH, (h + 1) * DH)
        o_acc[rows, cols] = (
            o_acc[rows, cols] * a_loc[:, h:h + 1]
            + o_rx_ref[:, cols].astype(jnp.float32) * a_rx[:, h:h + 1])
    ml_acc[0, rows, :] = m_new
    ml_acc[1, rows, :] = l_loc * a_loc + l_rx * a_rx


def _ring_body(o_ref, m_ref, l_ref, wo_ref, out_ref,
               o_cacc, ml_cacc, o_rx, ml_rx, ag_buf,
               rs_ssems, rs_rsems, ag_ssems, ag_rsems):
    p = lax.axis_index("i")

    barrier = pltpu.get_barrier_semaphore()
    for d in range(1, N_DEV):
        pl.semaphore_signal(barrier, inc=1,
                            device_id=(lax.rem(p + d, N_DEV),),
                            device_id_type=pl.DeviceIdType.MESH)
    pl.semaphore_wait(barrier, N_DEV - 1)

    rdmas = []
    for d in range(1, N_DEV):
        c = lax.rem(p - d + N_DEV, N_DEV)
        o_rdma = pltpu.make_async_remote_copy(
            src_ref=o_ref.at[pl.ds(c * CHUNK, CHUNK), :],
            dst_ref=o_rx.at[d - 1],
            send_sem=rs_ssems.at[0, d],
            recv_sem=rs_rsems.at[0, d - 1],
            device_id=(c,), device_id_type=pl.DeviceIdType.MESH)
        m_rdma = pltpu.make_async_remote_copy(
            src_ref=m_ref.at[pl.ds(c * CHUNK, CHUNK), :],
            dst_ref=ml_rx.at[d - 1, 0],
            send_sem=rs_ssems.at[1, d],
            recv_sem=rs_rsems.at[1, d - 1],
            device_id=(c,), device_id_type=pl.DeviceIdType.MESH)
        l_rdma = pltpu.make_async_remote_copy(
            src_ref=l_ref.at[pl.ds(c * CHUNK, CHUNK), :],
            dst_ref=ml_rx.at[d - 1, 1],
            send_sem=rs_ssems.at[2, d],
            recv_sem=rs_rsems.at[2, d - 1],
            device_id=(c,), device_id_type=pl.DeviceIdType.MESH)
        o_rdma.start()
        m_rdma.start()
        l_rdma.start()
        rdmas += [o_rdma, m_rdma, l_rdma]

    own = pl.ds(p * CHUNK, CHUNK)
    o_cacc[:, :] = o_ref[own, :].astype(jnp.float32)
    ml_cacc[0, :, :] = m_ref[own, :]
    ml_cacc[1, :, :] = l_ref[own, :]

    for j in range(N_DEV - 1):
        rdmas[3 * j].wait_recv()
        rdmas[3 * j + 1].wait_recv()
        rdmas[3 * j + 2].wait_recv()
        m_loc = ml_cacc[0, :, :]
        l_loc = ml_cacc[1, :, :]
        m_rx_v = ml_rx[j, 0]
        l_rx_v = ml_rx[j, 1]
        m_new = jnp.maximum(m_loc, m_rx_v)
        a_loc = jnp.exp(m_loc - m_new)
        a_rx = jnp.exp(m_rx_v - m_new)
        for h in range(HQ):
            cols = slice(h * DH, (h + 1) * DH)
            o_cacc[:, cols] = (
                o_cacc[:, cols] * a_loc[:, h:h + 1]
                + o_rx[j][:, cols].astype(jnp.float32) * a_rx[:, h:h + 1])
        ml_cacc[0, :, :] = m_new
        ml_cacc[1, :, :] = l_loc * a_loc + l_rx_v * a_rx

    l_own = ml_cacc[1, :, :]
    ctx_cols = []
    for h in range(HQ):
        cols = slice(h * DH, (h + 1) * DH)
        ctx_cols.append(o_cacc[:, cols] / l_own[:, h:h + 1])
    ctx = jnp.concatenate(ctx_cols, axis=1)
    ag_buf[own, :] = jnp.dot(
        ctx, wo_ref[:, :],
        preferred_element_type=jnp.float32).astype(jnp.bfloat16)

    for d in range(1, N_DEV):
        tgt = lax.rem(p + d, N_DEV)
        rdma = pltpu.make_async_remote_copy(
            src_ref=ag_buf.at[own, :],
            dst_ref=ag_buf.at[own, :],
            send_sem=ag_ssems.at[d],
            recv_sem=ag_rsems.at[p],
            device_id=(tgt,), device_id_type=pl.DeviceIdType.MESH)
        rdma.start()
        rdmas.append(rdma)
    for d in range(1, N_DEV):
        c = lax.rem(p + d, N_DEV)
        dst = ag_buf.at[pl.ds(c * CHUNK, CHUNK), :]
        rx = pltpu.make_async_remote_copy(
            src_ref=dst, dst_ref=dst,
            send_sem=ag_ssems.at[0],
            recv_sem=ag_rsems.at[c],
            device_id=(p,), device_id_type=pl.DeviceIdType.MESH)
        rx.wait_recv()

    out_ref[0, :, :] = ag_buf[:, :].astype(jnp.float32)

    for r in rdmas:
        r.wait_send()


_SRC = [(0, 11, 22)[b % 3] + b // 3 for b in range(SQ // BLK)]


def kernel(x, Wq, K_ext, V_ext, Wo):
    k2 = K_ext.reshape(SKV_LOC, HQ * DH)
    v2 = V_ext.reshape(SKV_LOC, HQ * DH)
    x4 = x.reshape(1, SQ // BLK, BLK, D)
    xp = jnp.concatenate(
        [x4[:, 0::3], x4[:, 1::3], x4[:, 2::3], x4[:, -1:]],
        axis=1).reshape(1, SQP, D)
    o, m, l = pl.pallas_call(
        _attn_body,
        grid=(HQ,),
        in_specs=[
            pl.BlockSpec((1, SQP, D), lambda h: (0, 0, 0)),
            pl.BlockSpec((D, DH), lambda h: (0, h)),
            pl.BlockSpec((SKV_LOC, DH), lambda h: (0, h)),
            pl.BlockSpec((SKV_LOC, DH), lambda h: (0, h)),
        ],
        out_specs=[
            pl.BlockSpec((SQP, DH), lambda h: (0, h)),
            pl.BlockSpec((SQP, HQ), lambda h: (0, 0)),
            pl.BlockSpec((SQP, HQ), lambda h: (0, 0)),
        ],
        out_shape=[
            jax.ShapeDtypeStruct((SQP, D), jnp.bfloat16),
            jax.ShapeDtypeStruct((SQP, HQ), jnp.float32),
            jax.ShapeDtypeStruct((SQP, HQ), jnp.float32),
        ],
        scratch_shapes=[
            pltpu.VMEM((CLS, DH), jnp.float32),
            pltpu.VMEM((CLS, DH), jnp.float32),
        ],
    )(xp, Wq, k2, v2)

    out_perm = pl.pallas_call(
        _ring_body,
        out_shape=jax.ShapeDtypeStruct((1, SQ, D), jnp.float32),
        in_specs=[pl.BlockSpec(memory_space=pltpu.VMEM)] * 4,
        out_specs=pl.BlockSpec(memory_space=pltpu.VMEM),
        scratch_shapes=[
            pltpu.VMEM((CHUNK, D), jnp.float32),
            pltpu.VMEM((2, CHUNK, HQ), jnp.float32),
            pltpu.VMEM((N_DEV - 1, CHUNK, D), jnp.bfloat16),
            pltpu.VMEM((N_DEV - 1, 2, CHUNK, HQ), jnp.float32),
            pltpu.VMEM((SQ, D), jnp.bfloat16),
            pltpu.SemaphoreType.DMA((3, N_DEV)),
            pltpu.SemaphoreType.DMA((3, N_DEV)),
            pltpu.SemaphoreType.DMA((N_DEV,)),
            pltpu.SemaphoreType.DMA((N_DEV,)),
        ],
        compiler_params=pltpu.CompilerParams(collective_id=0),
    )(o, m, l, Wo)
    o4 = out_perm.reshape(1, SQ // BLK, BLK, D)
    return jnp.concatenate([o4[:, s:s + 1] for s in _SRC],
                           axis=1).reshape(1, SQ, D)


# device time: 164439 ns/iter; 1.4053x vs baseline; 1.0067x over previous
import jax
import jax.numpy as jnp
from jax import lax
from jax.experimental import pallas as pl
from jax.experimental.pallas import tpu as pltpu

N_DEV = 8
SQ = 2048
D = 1024
HQ = 8
DH = 128
SKV_LOC = 2048
BLK = 64
CHUNK = SQ // N_DEV
QT = 512
N_QT = SQ // QT
SCALE = 0.08838834764831843
NEG = -1e9


CLS = 11 * BLK
SQP = 3 * CLS


def _attn_body(x_ref, wq_ref, k_ref, v_ref, o_ref, m_ref, l_ref, kc, vc):
    p = lax.axis_index("i")
    h = pl.program_id(0)
    is_p0 = p == 0
    for r in range(3):
        seg = r * CLS
        q = jnp.dot(x_ref[0, seg:seg + CLS, :], wq_ref[:, :],
                    preferred_element_type=jnp.float32)
        c = (3 - r) % 3
        b0 = lax.rem(jnp.int32(c) - 2 * p + 48, 3)
        for j in range(11):
            start = jnp.minimum(b0 * BLK + j * 3 * BLK, SKV_LOC - BLK)
            kc[j * BLK:(j + 1) * BLK, :] = k_ref[pl.ds(start, BLK), :]
            vc[j * BLK:(j + 1) * BLK, :] = v_ref[pl.ds(start, BLK), :]
        nv = jnp.where(b0 == 2, 10 * BLK, 11 * BLK)
        s = lax.dot_general(q, kc[:, :], (((1,), (1,)), ((), ())),
                            preferred_element_type=jnp.float32) * SCALE
        col = lax.broadcasted_iota(jnp.int32, (1, CLS), 1)
        s = jnp.where(col < nv, s, NEG)
        m = jnp.max(s, axis=1, keepdims=True)
        if r == 0:
            w = jnp.exp(s - m)
            lsum = jnp.sum(w, axis=1, keepdims=True)
            o = lax.dot_general(w, vc[:, :], (((1,), (0,)), ((), ())),
                                preferred_element_type=jnp.float32)
        else:
            s0 = lax.dot_general(q, k_ref[0:BLK, :],
                                 (((1,), (1,)), ((), ())),
                                 preferred_element_type=jnp.float32) * SCALE
            s0 = jnp.where(is_p0, s0, NEG)
            q3 = q.reshape(11, BLK, DH)
            kd = jnp.stack([k_ref[min(r + 3 * j, 31) * BLK:
                                  (min(r + 3 * j, 31) + 1) * BLK, :]
                            for j in range(11)])
            vd = jnp.stack([v_ref[min(r + 3 * j, 31) * BLK:
                                  (min(r + 3 * j, 31) + 1) * BLK, :]
                            for j in range(11)])
            sd = lax.dot_general(q3, kd, (((2,), (2,)), ((0,), (0,))),
                                 preferred_element_type=jnp.float32) * SCALE
            sd = jnp.where(is_p0, sd, NEG).reshape(CLS, BLK)
            m = jnp.maximum(
                m, jnp.maximum(jnp.max(s0, axis=1, keepdims=True),
                               jnp.max(sd, axis=1, keepdims=True)))
            w = jnp.exp(s - m)
            w0 = jnp.exp(s0 - m)
            wd = jnp.exp(sd - m)
            lsum = (jnp.sum(w, axis=1, keepdims=True)
                    + jnp.sum(w0, axis=1, keepdims=True)
                    + jnp.sum(wd, axis=1, keepdims=True))
            od = lax.dot_general(wd.reshape(11, BLK, BLK), vd,
                                 (((2,), (1,)), ((0,), (0,))),
                                 preferred_element_type=jnp.float32)
            o = (lax.dot_general(w, vc[:, :], (((1,), (0,)), ((), ())),
                                 preferred_element_type=jnp.float32)
                 + lax.dot_general(w0, v_ref[0:BLK, :],
                                   (((1,), (0,)), ((), ())),
                                   preferred_element_type=jnp.float32)
                 + od.reshape(CLS, DH))
        o_ref[seg:seg + CLS, :] = o.astype(jnp.bfloat16)
        lane = lax.broadcasted_iota(jnp.int32, (CLS, HQ), 1)
        m_ref[seg:seg + CLS, :] = jnp.where(
            lane == h, m, m_ref[seg:seg + CLS, :])
        l_ref[seg:seg + CLS, :] = jnp.where(
            lane == h, lsum, l_ref[seg:seg + CLS, :])


HALF = CHUNK // 2


def _merge(o_acc, ml_acc, o_rx_ref, ml_rx_ref, rows):
    m_loc = ml_acc[0, rows, :]
    l_loc = ml_acc[1, rows, :]
    m_rx = ml_rx_ref[0]
    l_rx = ml_rx_ref[1]
    m_new = jnp.maximum(m_loc, m_rx)
    a_loc = jnp.exp(m_loc - m_new)
    a_rx = jnp.exp(m_rx - m_new)
    for h in range(HQ):
        cols = slice(h * DH, (h + 1) * DH)
        o_acc[rows, cols] = (
            o_acc[rows, cols] * a_loc[:, h:h + 1]
            + o_rx_ref[:, cols].astype(jnp.float32) * a_rx[:, h:h + 1])
    ml_acc[0, rows, :] = m_new
    ml_acc[1, rows, :] = l_loc * a_loc + l_rx * a_rx


def _ring_body(o_ref, m_ref, l_ref, wo_ref, out_ref,
               o_cacc, ml_cacc, o_rx, ml_rx, ag_buf,
               rs_ssems, rs_rsems, ag_ssems, ag_rsems):
    p = lax.axis_index("i")

    barrier = pltpu.get_barrier_semaphore()
    for d in range(1, N_DEV):
        pl.semaphore_signal(barrier, inc=1,
                            device_id=(lax.rem(p + d, N_DEV),),
                            device_id_type=pl.DeviceIdType.MESH)
    pl.semaphore_wait(barrier, N_DEV - 1)

    rdmas = []
    for d in range(1, N_DEV):
        c = lax.rem(p - d + N_DEV, N_DEV)
        o_rdma = pltpu.make_async_remote_copy(
            src_ref=o_ref.at[pl.ds(c * CHUNK, CHUNK), :],
            dst_ref=o_rx.at[d - 1],
            send_sem=rs_ssems.at[0, d],
            recv_sem=rs_rsems.at[0, d - 1],
            device_id=(c,), device_id_type=pl.DeviceIdType.MESH)
        m_rdma = pltpu.make_async_remote_copy(
            src_ref=m_ref.at[pl.ds(c * CHUNK, CHUNK), :],
            dst_ref=ml_rx.at[d - 1, 0],
            send_sem=rs_ssems.at[1, d],
            recv_sem=rs_rsems.at[1, d - 1],
            device_id=(c,), device_id_type=pl.DeviceIdType.MESH)
        l_rdma = pltpu.make_async_remote_copy(
            src_ref=l_ref.at[pl.ds(c * CHUNK, CHUNK), :],
            dst_ref=ml_rx.at[d - 1, 1],
            send_sem=rs_ssems.at[2, d],
            recv_sem=rs_rsems.at[2, d - 1],
            device_id=(c,), device_id_type=pl.DeviceIdType.MESH)
        o_rdma.start()
        m_rdma.start()
        l_rdma.start()
        rdmas += [o_rdma, m_rdma, l_rdma]

    own = pl.ds(p * CHUNK, CHUNK)
    o_cacc[:, :] = o_ref[own, :].astype(jnp.float32)
    ml_cacc[0, :, :] = m_ref[own, :]
    ml_cacc[1, :, :] = l_ref[own, :]

    for j in range(N_DEV - 1):
        rdmas[3 * j].wait_recv()
        rdmas[3 * j + 1].wait_recv()
        rdmas[3 * j + 2].wait_recv()
        m_loc = ml_cacc[0, :, :]
        l_loc = ml_cacc[1, :, :]
        m_rx_v = ml_rx[j, 0]
        l_rx_v = ml_rx[j, 1]
        m_new = jnp.maximum(m_loc, m_rx_v)
        a_loc = jnp.exp(m_loc - m_new)
        a_rx = jnp.exp(m_rx_v - m_new)
        for h in range(HQ):
            cols = slice(h * DH, (h + 1) * DH)
            o_cacc[:, cols] = (
                o_cacc[:, cols] * a_loc[:, h:h + 1]
                + o_rx[j][:, cols].astype(jnp.float32) * a_rx[:, h:h + 1])
        ml_cacc[0, :, :] = m_new
        ml_cacc[1, :, :] = l_loc * a_loc + l_rx_v * a_rx

    l_own = ml_cacc[1, :, :]
    ctx_cols = []
    for h in range(HQ):
        cols = slice(h * DH, (h + 1) * DH)
        ctx_cols.append(o_cacc[:, cols] / l_own[:, h:h + 1])
    ctx = jnp.concatenate(ctx_cols, axis=1)
    ag_buf[own, :] = jnp.dot(
        ctx, wo_ref[:, :],
        preferred_element_type=jnp.float32).astype(jnp.bfloat16)

    for d in range(1, N_DEV):
        tgt = lax.rem(p + d, N_DEV)
        rdma = pltpu.make_async_remote_copy(
            src_ref=ag_buf.at[own, :],
            dst_ref=ag_buf.at[own, :],
            send_sem=ag_ssems.at[d],
            recv_sem=ag_rsems.at[p],
            device_id=(tgt,), device_id_type=pl.DeviceIdType.MESH)
        rdma.start()
        rdmas.append(rdma)
    for d in range(1, N_DEV):
        c = lax.rem(p + d, N_DEV)
        dst = ag_buf.at[pl.ds(c * CHUNK, CHUNK), :]
        rx = pltpu.make_async_remote_copy(
            src_ref=dst, dst_ref=dst,
            send_sem=ag_ssems.at[0],
            recv_sem=ag_rsems.at[c],
            device_id=(p,), device_id_type=pl.DeviceIdType.MESH)
        rx.wait_recv()

    for b in range(SQ // BLK):
        s = _SRC[b]
        out_ref[0, b * BLK:(b + 1) * BLK, :] = ag_buf[
            s * BLK:(s + 1) * BLK, :].astype(jnp.float32)

    for r in rdmas:
        r.wait_send()


_SRC = [(0, 11, 22)[b % 3] + b // 3 for b in range(SQ // BLK)]


def kernel(x, Wq, K_ext, V_ext, Wo):
    k2 = K_ext.reshape(SKV_LOC, HQ * DH)
    v2 = V_ext.reshape(SKV_LOC, HQ * DH)
    x4 = x.reshape(1, SQ // BLK, BLK, D)
    xp = jnp.concatenate(
        [x4[:, 0::3], x4[:, 1::3], x4[:, 2::3], x4[:, -1:]],
        axis=1).reshape(1, SQP, D)
    o, m, l = pl.pallas_call(
        _attn_body,
        grid=(HQ,),
        in_specs=[
            pl.BlockSpec((1, SQP, D), lambda h: (0, 0, 0)),
            pl.BlockSpec((D, DH), lambda h: (0, h)),
            pl.BlockSpec((SKV_LOC, DH), lambda h: (0, h)),
            pl.BlockSpec((SKV_LOC, DH), lambda h: (0, h)),
        ],
        out_specs=[
            pl.BlockSpec((SQP, DH), lambda h: (0, h)),
            pl.BlockSpec((SQP, HQ), lambda h: (0, 0)),
            pl.BlockSpec((SQP, HQ), lambda h: (0, 0)),
        ],
        out_shape=[
            jax.ShapeDtypeStruct((SQP, D), jnp.bfloat16),
            jax.ShapeDtypeStruct((SQP, HQ), jnp.float32),
            jax.ShapeDtypeStruct((SQP, HQ), jnp.float32),
        ],
        scratch_shapes=[
            pltpu.VMEM((CLS, DH), jnp.float32),
            pltpu.VMEM((CLS, DH), jnp.float32),
        ],
    )(xp, Wq, k2, v2)

    out_perm = pl.pallas_call(
        _ring_body,
        out_shape=jax.ShapeDtypeStruct((1, SQ, D), jnp.float32),
        in_specs=[pl.BlockSpec(memory_space=pltpu.VMEM)] * 4,
        out_specs=pl.BlockSpec(memory_space=pltpu.VMEM),
        scratch_shapes=[
            pltpu.VMEM((CHUNK, D), jnp.float32),
            pltpu.VMEM((2, CHUNK, HQ), jnp.float32),
            pltpu.VMEM((N_DEV - 1, CHUNK, D), jnp.bfloat16),
            pltpu.VMEM((N_DEV - 1, 2, CHUNK, HQ), jnp.float32),
            pltpu.VMEM((SQ, D), jnp.bfloat16),
            pltpu.SemaphoreType.DMA((3, N_DEV)),
            pltpu.SemaphoreType.DMA((3, N_DEV)),
            pltpu.SemaphoreType.DMA((N_DEV,)),
            pltpu.SemaphoreType.DMA((N_DEV,)),
        ],
        compiler_params=pltpu.CompilerParams(collective_id=0),
    )(o, m, l, Wo)
    return out_perm


# device time: 157187 ns/iter; 1.4702x vs baseline; 1.0461x over previous
import jax
import jax.numpy as jnp
from jax import lax
from jax.experimental import pallas as pl
from jax.experimental.pallas import tpu as pltpu

N_DEV = 8
SQ = 2048
D = 1024
HQ = 8
DH = 128
SKV_LOC = 2048
BLK = 64
CHUNK = SQ // N_DEV
QT = 512
N_QT = SQ // QT
SCALE = 0.08838834764831843
NEG = -1e9


CLS = 11 * BLK
SQP = 3 * CLS
_QB_ORDER = [b for r in range(3) for b in range(SQ // BLK) if b % 3 == r]


def _attn_body(x_ref, wq_ref, k_ref, v_ref, o_ref, m_ref, l_ref,
               kc, vc, xp):
    p = lax.axis_index("i")
    h = pl.program_id(0)
    is_p0 = p == 0

    @pl.when(h == 0)
    def _():
        for pos, b in enumerate(_QB_ORDER + [31]):
            xp[pos * BLK:(pos + 1) * BLK, :] = x_ref[
                0, b * BLK:(b + 1) * BLK, :]

    for r in range(3):
        seg = r * CLS
        q = jnp.dot(xp[seg:seg + CLS, :], wq_ref[:, :],
                    preferred_element_type=jnp.float32)
        c = (3 - r) % 3
        b0 = lax.rem(jnp.int32(c) - 2 * p + 48, 3)
        for j in range(11):
            start = jnp.minimum(b0 * BLK + j * 3 * BLK, SKV_LOC - BLK)
            kc[j * BLK:(j + 1) * BLK, :] = k_ref[pl.ds(start, BLK), :]
            vc[j * BLK:(j + 1) * BLK, :] = v_ref[pl.ds(start, BLK), :]
        nv = jnp.where(b0 == 2, 10 * BLK, 11 * BLK)
        s = lax.dot_general(q, kc[:, :], (((1,), (1,)), ((), ())),
                            preferred_element_type=jnp.float32) * SCALE
        col = lax.broadcasted_iota(jnp.int32, (1, CLS), 1)
        s = jnp.where(col < nv, s, NEG)
        m = jnp.max(s, axis=1, keepdims=True)
        if r == 0:
            w = jnp.exp(s - m)
            lsum = jnp.sum(w, axis=1, keepdims=True)
            o = lax.dot_general(w, vc[:, :], (((1,), (0,)), ((), ())),
                                preferred_element_type=jnp.float32)
        else:
            s0 = lax.dot_general(q, k_ref[0:BLK, :],
                                 (((1,), (1,)), ((), ())),
                                 preferred_element_type=jnp.float32) * SCALE
            s0 = jnp.where(is_p0, s0, NEG)
            q3 = q.reshape(11, BLK, DH)
            kd = jnp.stack([k_ref[min(r + 3 * j, 31) * BLK:
                                  (min(r + 3 * j, 31) + 1) * BLK, :]
                            for j in range(11)])
            vd = jnp.stack([v_ref[min(r + 3 * j, 31) * BLK:
                                  (min(r + 3 * j, 31) + 1) * BLK, :]
                            for j in range(11)])
            sd = lax.dot_general(q3, kd, (((2,), (2,)), ((0,), (0,))),
                                 preferred_element_type=jnp.float32) * SCALE
            sd = jnp.where(is_p0, sd, NEG).reshape(CLS, BLK)
            m = jnp.maximum(
                m, jnp.maximum(jnp.max(s0, axis=1, keepdims=True),
                               jnp.max(sd, axis=1, keepdims=True)))
            w = jnp.exp(s - m)
            w0 = jnp.exp(s0 - m)
            wd = jnp.exp(sd - m)
            lsum = (jnp.sum(w, axis=1, keepdims=True)
                    + jnp.sum(w0, axis=1, keepdims=True)
                    + jnp.sum(wd, axis=1, keepdims=True))
            od = lax.dot_general(wd.reshape(11, BLK, BLK), vd,
                                 (((2,), (1,)), ((0,), (0,))),
                                 preferred_element_type=jnp.float32)
            o = (lax.dot_general(w, vc[:, :], (((1,), (0,)), ((), ())),
                                 preferred_element_type=jnp.float32)
                 + lax.dot_general(w0, v_ref[0:BLK, :],
                                   (((1,), (0,)), ((), ())),
                                   preferred_element_type=jnp.float32)
                 + od.reshape(CLS, DH))
        o_ref[seg:seg + CLS, :] = o.astype(jnp.bfloat16)
        lane = lax.broadcasted_iota(jnp.int32, (CLS, HQ), 1)
        m_ref[seg:seg + CLS, :] = jnp.where(
            lane == h, m, m_ref[seg:seg + CLS, :])
        l_ref[seg:seg + CLS, :] = jnp.where(
            lane == h, lsum, l_ref[seg:seg + CLS, :])


HALF = CHUNK // 2


def _merge(o_acc, ml_acc, o_rx_ref, ml_rx_ref, rows):
    m_loc = ml_acc[0, rows, :]
    l_loc = ml_acc[1, rows, :]
    m_rx = ml_rx_ref[0]
    l_rx = ml_rx_ref[1]
    m_new = jnp.maximum(m_loc, m_rx)
    a_loc = jnp.exp(m_loc - m_new)
    a_rx = jnp.exp(m_rx - m_new)
    for h in range(HQ):
        cols = slice(h * DH, (h + 1) * DH)
        o_acc[rows, cols] = (
            o_acc[rows, cols] * a_loc[:, h:h + 1]
            + o_rx_ref[:, cols].astype(jnp.float32) * a_rx[:, h:h + 1])
    ml_acc[0, rows, :] = m_new
    ml_acc[1, rows, :] = l_loc * a_loc + l_rx * a_rx


def _ring_body(o_ref, m_ref, l_ref, wo_ref, out_ref,
               o_cacc, ml_cacc, o_rx, ml_rx, ag_buf,
               rs_ssems, rs_rsems, ag_ssems, ag_rsems):
    p = lax.axis_index("i")

    barrier = pltpu.get_barrier_semaphore()
    for d in range(1, N_DEV):
        pl.semaphore_signal(barrier, inc=1,
                            device_id=(lax.rem(p + d, N_DEV),),
                            device_id_type=pl.DeviceIdType.MESH)
    pl.semaphore_wait(barrier, N_DEV - 1)

    rdmas = []
    for d in range(1, N_DEV):
        c = lax.rem(p - d + N_DEV, N_DEV)
        o_rdma = pltpu.make_async_remote_copy(
            src_ref=o_ref.at[pl.ds(c * CHUNK, CHUNK), :],
            dst_ref=o_rx.at[d - 1],
            send_sem=rs_ssems.at[0, d],
            recv_sem=rs_rsems.at[0, d - 1],
            device_id=(c,), device_id_type=pl.DeviceIdType.MESH)
        m_rdma = pltpu.make_async_remote_copy(
            src_ref=m_ref.at[pl.ds(c * CHUNK, CHUNK), :],
            dst_ref=ml_rx.at[d - 1, 0],
            send_sem=rs_ssems.at[1, d],
            recv_sem=rs_rsems.at[1, d - 1],
            device_id=(c,), device_id_type=pl.DeviceIdType.MESH)
        l_rdma = pltpu.make_async_remote_copy(
            src_ref=l_ref.at[pl.ds(c * CHUNK, CHUNK), :],
            dst_ref=ml_rx.at[d - 1, 1],
            send_sem=rs_ssems.at[2, d],
            recv_sem=rs_rsems.at[2, d - 1],
            device_id=(c,), device_id_type=pl.DeviceIdType.MESH)
        o_rdma.start()
        m_rdma.start()
        l_rdma.start()
        rdmas += [o_rdma, m_rdma, l_rdma]

    own = pl.ds(p * CHUNK, CHUNK)
    o_cacc[:, :] = o_ref[own, :].astype(jnp.float32)
    ml_cacc[0, :, :] = m_ref[own, :]
    ml_cacc[1, :, :] = l_ref[own, :]

    for j in range(N_DEV - 1):
        rdmas[3 * j].wait_recv()
        rdmas[3 * j + 1].wait_recv()
        rdmas[3 * j + 2].wait_recv()
        m_loc = ml_cacc[0, :, :]
        l_loc = ml_cacc[1, :, :]
        m_rx_v = ml_rx[j, 0]
        l_rx_v = ml_rx[j, 1]
        m_new = jnp.maximum(m_loc, m_rx_v)
        a_loc = jnp.exp(m_loc - m_new)
        a_rx = jnp.exp(m_rx_v - m_new)
        for h in range(HQ):
            cols = slice(h * DH, (h + 1) * DH)
            o_cacc[:, cols] = (
                o_cacc[:, cols] * a_loc[:, h:h + 1]
                + o_rx[j][:, cols].astype(jnp.float32) * a_rx[:, h:h + 1])
        ml_cacc[0, :, :] = m_new
        ml_cacc[1, :, :] = l_loc * a_loc + l_rx_v * a_rx

    l_own = ml_cacc[1, :, :]
    ctx_cols = []
    for h in range(HQ):
        cols = slice(h * DH, (h + 1) * DH)
        ctx_cols.append(o_cacc[:, cols] / l_own[:, h:h + 1])
    ctx = jnp.concatenate(ctx_cols, axis=1)
    ag_buf[own, :] = jnp.dot(
        ctx, wo_ref[:, :],
        preferred_element_type=jnp.float32).astype(jnp.bfloat16)

    for d in range(1, N_DEV):
        tgt = lax.rem(p + d, N_DEV)
        rdma = pltpu.make_async_remote_copy(
            src_ref=ag_buf.at[own, :],
            dst_ref=ag_buf.at[own, :],
            send_sem=ag_ssems.at[d],
            recv_sem=ag_rsems.at[p],
            device_id=(tgt,), device_id_type=pl.DeviceIdType.MESH)
        rdma.start()
        rdmas.append(rdma)
    for d in range(1, N_DEV):
        c = lax.rem(p + d, N_DEV)
        dst = ag_buf.at[pl.ds(c * CHUNK, CHUNK), :]
        rx = pltpu.make_async_remote_copy(
            src_ref=dst, dst_ref=dst,
            send_sem=ag_ssems.at[0],
            recv_sem=ag_rsems.at[c],
            device_id=(p,), device_id_type=pl.DeviceIdType.MESH)
        rx.wait_recv()

    for b in range(SQ // BLK):
        s = _SRC[b]
        out_ref[0, b * BLK:(b + 1) * BLK, :] = ag_buf[
            s * BLK:(s + 1) * BLK, :].astype(jnp.float32)

    for r in rdmas:
        r.wait_send()


_SRC = [(0, 11, 22)[b % 3] + b // 3 for b in range(SQ // BLK)]


def kernel(x, Wq, K_ext, V_ext, Wo):
    k2 = K_ext.reshape(SKV_LOC, HQ * DH)
    v2 = V_ext.reshape(SKV_LOC, HQ * DH)
    o, m, l = pl.pallas_call(
        _attn_body,
        grid=(HQ,),
        in_specs=[
            pl.BlockSpec((1, SQ, D), lambda h: (0, 0, 0)),
            pl.BlockSpec((D, DH), lambda h: (0, h)),
            pl.BlockSpec((SKV_LOC, DH), lambda h: (0, h)),
            pl.BlockSpec((SKV_LOC, DH), lambda h: (0, h)),
        ],
        out_specs=[
            pl.BlockSpec((SQP, DH), lambda h: (0, h)),
            pl.BlockSpec((SQP, HQ), lambda h: (0, 0)),
            pl.BlockSpec((SQP, HQ), lambda h: (0, 0)),
        ],
        out_shape=[
            jax.ShapeDtypeStruct((SQP, D), jnp.bfloat16),
            jax.ShapeDtypeStruct((SQP, HQ), jnp.float32),
            jax.ShapeDtypeStruct((SQP, HQ), jnp.float32),
        ],
        scratch_shapes=[
            pltpu.VMEM((CLS, DH), jnp.float32),
            pltpu.VMEM((CLS, DH), jnp.float32),
            pltpu.VMEM((SQP, D), jnp.float32),
        ],
    )(x, Wq, k2, v2)

    out_perm = pl.pallas_call(
        _ring_body,
        out_shape=jax.ShapeDtypeStruct((1, SQ, D), jnp.float32),
        in_specs=[pl.BlockSpec(memory_space=pltpu.VMEM)] * 4,
        out_specs=pl.BlockSpec(memory_space=pltpu.VMEM),
        scratch_shapes=[
            pltpu.VMEM((CHUNK, D), jnp.float32),
            pltpu.VMEM((2, CHUNK, HQ), jnp.float32),
            pltpu.VMEM((N_DEV - 1, CHUNK, D), jnp.bfloat16),
            pltpu.VMEM((N_DEV - 1, 2, CHUNK, HQ), jnp.float32),
            pltpu.VMEM((SQ, D), jnp.bfloat16),
            pltpu.SemaphoreType.DMA((3, N_DEV)),
            pltpu.SemaphoreType.DMA((3, N_DEV)),
            pltpu.SemaphoreType.DMA((N_DEV,)),
            pltpu.SemaphoreType.DMA((N_DEV,)),
        ],
        compiler_params=pltpu.CompilerParams(collective_id=0),
    )(o, m, l, Wo)
    return out_perm
